# Initial kernel scaffold; baseline (speedup 1.0000x reference)
import jax
import jax.numpy as jnp
from jax.experimental import pallas as pl


def kernel(boxes, scores, idxs):
    raise NotImplementedError("write your pallas kernel here")


# speedup vs baseline: 151.4042x; 151.4042x over previous
import functools

import jax
import jax.numpy as jnp
from jax.experimental import pallas as pl
from jax.experimental.pallas import tpu as pltpu

N = 5000
TILE = 128
P = 5120
T_TILES = P // TILE
IOU_THRESH = 0.5


def _pair_sup(x1c, y1c, x2c, y2c, ac, cc, x1i, y1i, x2i, y2i, ai, ci):
    shp = (TILE, TILE)
    xx1 = jnp.maximum(jnp.broadcast_to(x1c, shp), jnp.broadcast_to(x1i, shp))
    yy1 = jnp.maximum(jnp.broadcast_to(y1c, shp), jnp.broadcast_to(y1i, shp))
    xx2 = jnp.minimum(jnp.broadcast_to(x2c, shp), jnp.broadcast_to(x2i, shp))
    yy2 = jnp.minimum(jnp.broadcast_to(y2c, shp), jnp.broadcast_to(y2i, shp))
    w = jnp.maximum(0.0, xx2 - xx1 + 1.0)
    h = jnp.maximum(0.0, yy2 - yy1 + 1.0)
    inter = w * h
    union = jnp.broadcast_to(ac, shp) + jnp.broadcast_to(ai, shp) - inter
    iou = inter / union
    sup = jnp.logical_and(iou > IOU_THRESH,
                          jnp.broadcast_to(cc, shp) == jnp.broadcast_to(ci, shp))
    return sup.astype(jnp.float32)


def _nms_body(jstart_ref,
              x1c_ref, y1c_ref, x2c_ref, y2c_ref, ac_ref, cc_ref,
              x1r_ref, y1r_ref, x2r_ref, y2r_ref, ar_ref, cr_ref,
              keep_ref):
    ident = (jax.lax.broadcasted_iota(jnp.int32, (TILE, TILE), 0) ==
             jax.lax.broadcasted_iota(jnp.int32, (TILE, TILE), 1)
             ).astype(jnp.float32)
    tri = (jax.lax.broadcasted_iota(jnp.int32, (TILE, TILE), 0) <
           jax.lax.broadcasted_iota(jnp.int32, (TILE, TILE), 1)
           ).astype(jnp.float32)

    def col_slices(j0):
        return (x1c_ref[pl.ds(j0, TILE), :], y1c_ref[pl.ds(j0, TILE), :],
                x2c_ref[pl.ds(j0, TILE), :], y2c_ref[pl.ds(j0, TILE), :],
                ac_ref[pl.ds(j0, TILE), :], cc_ref[pl.ds(j0, TILE), :])

    def tile_step(t, carry):
        i0 = t * TILE
        row = (x1r_ref[:, pl.ds(i0, TILE)], y1r_ref[:, pl.ds(i0, TILE)],
               x2r_ref[:, pl.ds(i0, TILE)], y2r_ref[:, pl.ds(i0, TILE)],
               ar_ref[:, pl.ds(i0, TILE)], cr_ref[:, pl.ds(i0, TILE)])

        def ext_step(jt, acc):
            col = col_slices(jt * TILE)
            sup = _pair_sup(*col, *row)
            keep_j = jnp.broadcast_to(keep_ref[pl.ds(jt * TILE, TILE), :],
                                      (TILE, TILE))
            return jnp.maximum(acc, jnp.max(sup * keep_j, axis=0,
                                            keepdims=True))

        acc = jax.lax.fori_loop(jstart_ref[t], t, ext_step,
                                jnp.zeros((1, TILE), jnp.float32))

        col_t = col_slices(i0)
        d_mat = _pair_sup(*col_t, *row) * tri

        def cond(c):
            return c[1]

        def lbody(c):
            k_col, _ = c
            sup_loc = jnp.max(d_mat * jnp.broadcast_to(k_col, (TILE, TILE)),
                              axis=0, keepdims=True)
            tot = jnp.maximum(acc, sup_loc)
            knew_row = jnp.where(tot > 0.0, 0.0, 1.0)
            knew_col = jax.lax.dot_general(
                ident, knew_row, (((1,), (1,)), ((), ())),
                preferred_element_type=jnp.float32)
            changed = jnp.any(knew_col != k_col)
            return (knew_col, changed)

        k_final, _ = jax.lax.while_loop(
            cond, lbody, (jnp.ones((TILE, 1), jnp.float32), True))
        keep_ref[pl.ds(i0, TILE), :] = k_final
        return carry

    jax.lax.fori_loop(0, T_TILES, tile_step, 0)


@jax.jit
def kernel(boxes, scores, idxs):
    max_coord = jnp.max(boxes)
    offsets = idxs.astype(boxes.dtype) * (max_coord + 1.0)
    b = boxes + offsets[:, None]
    x1, y1, x2, y2 = b[:, 0], b[:, 1], b[:, 2], b[:, 3]
    area = (x2 - x1 + 1.0) * (y2 - y1 + 1.0)

    ord1 = jnp.argsort(-scores)
    ord2 = jnp.argsort(idxs[ord1])
    order = ord1[ord2]

    pad = P - N
    pad_f = jnp.zeros((pad,), jnp.float32)

    def padded(v, pad_vals):
        return jnp.concatenate([v[order], pad_vals])

    x1s = padded(x1, pad_f)
    y1s = padded(y1, pad_f)
    x2s = padded(x2, pad_f)
    y2s = padded(y2, pad_f)
    areas = padded(area, jnp.ones((pad,), jnp.float32))
    clss = padded(idxs.astype(jnp.float32),
                  1000.0 + jnp.arange(pad, dtype=jnp.float32))

    counts = jnp.bincount(idxs, length=8)
    seg_start = jnp.concatenate(
        [jnp.zeros((1,), jnp.int32), jnp.cumsum(counts)[:-1].astype(jnp.int32)])
    tile_first_cls = jnp.clip(
        clss[:: TILE].astype(jnp.int32), 0, 7)
    jstart = seg_start[tile_first_cls] // TILE

    col = lambda v: v.reshape(P, 1)
    row = lambda v: v.reshape(1, P)

    keep = pl.pallas_call(
        _nms_body,
        out_shape=jax.ShapeDtypeStruct((P, 1), jnp.float32),
        in_specs=[pl.BlockSpec(memory_space=pltpu.SMEM)] +
                 [pl.BlockSpec(memory_space=pltpu.VMEM)] * 12,
        out_specs=pl.BlockSpec(memory_space=pltpu.VMEM),
    )(jstart,
      col(x1s), col(y1s), col(x2s), col(y2s), col(areas), col(clss),
      row(x1s), row(y1s), row(x2s), row(y2s), row(areas), row(clss))

    keep_sorted = keep[:N, 0] > 0.0
    scores_sorted = scores[order]
    out = jnp.zeros((N,), jnp.float32).at[order].set(
        jnp.where(keep_sorted, scores_sorted, 0.0))
    out = out.at[0].set(jnp.where(jnp.all(keep_sorted), out[0], 0.0))
    return out
